# baseline (speedup 1.0000x reference)
import jax
import jax.numpy as jnp
from jax.experimental import pallas as pl
from jax.experimental.pallas import tpu as pltpu


def _copy_body(ea_ref, eao_ref):
    eao_ref[...] = ea_ref[...]


def kernel(x, x_lstm, encoded_z_gnss, edge_index, edge_attr,
           node_indexes_related_to_agent, edge_indexes_related_to_agent):
    E, DE = edge_attr.shape
    LANES = 128
    ER = (E * DE) // LANES
    ea = edge_attr.reshape(ER, LANES)
    G = 10
    ean = pl.pallas_call(
        _copy_body,
        grid=(G,),
        in_specs=[pl.BlockSpec((ER // G, LANES), lambda i: (i, 0))],
        out_specs=pl.BlockSpec((ER // G, LANES), lambda i: (i, 0)),
        out_shape=jax.ShapeDtypeStruct((ER, LANES), edge_attr.dtype),
    )(ea)
    return (x, ean.reshape(E, DE))
